# baseline (speedup 1.0000x reference)
import functools

import jax
import jax.numpy as jnp
from jax import lax
from jax.experimental import pallas as pl
from jax.experimental.pallas import tpu as pltpu
from jax.experimental.pallas import tpu_sc as plsc

N = 10000
E = 320000
H = 128

NC = 2
NS = 16
NW = NC * NS
CH = 128
NCHUNK = 80
EPW = NCHUNK * CH
EPAD = NW * EPW
NPAD = 10240
ROWS_PT = NPAD // NS

_sc_mesh = plsc.VectorSubcoreMesh(core_axis_name="c", subcore_axis_name="s")


@functools.partial(
    pl.kernel,
    out_type=jax.ShapeDtypeStruct((NC, NPAD, H), jnp.float32),
    mesh=_sc_mesh,
    scratch_types=[
        pltpu.VMEM((NCHUNK, CH), jnp.int32),
        pltpu.VMEM((CH,), jnp.int32),
        pltpu.VMEM((CH,), jnp.int32),
        pltpu.VMEM((CH,), jnp.int32),
        pltpu.VMEM((CH,), jnp.int32),
        pltpu.VMEM((CH, H), jnp.float32),
        pltpu.VMEM((CH, H), jnp.float32),
        pltpu.VMEM_SHARED((NPAD, H), jnp.float32),
        pltpu.SemaphoreType.DMA,
        pltpu.SemaphoreType.DMA,
    ],
)
def _segsum(h_hbm, idx_hbm, zero_hbm, out_hbm,
            idx_v, src_a, dst_a, src_b, dst_b, rows_a, rows_b, acc_sh,
            sem_ga, sem_gb):
    c = lax.axis_index("c")
    s = lax.axis_index("s")
    wid = c * NS + s

    def unpack(i, src_ref, dst_ref):
        for k in range(CH // 16):
            v = idx_v[i, pl.ds(k * 16, 16)]
            src_ref[pl.ds(k * 16, 16)] = v & 0xFFFF
            dst_ref[pl.ds(k * 16, 16)] = lax.shift_right_logical(v, 16)

    r0 = s * ROWS_PT
    for z in range(ROWS_PT // CH):
        pltpu.async_copy(zero_hbm, acc_sh.at[pl.ds(r0 + z * CH, CH)], sem_ga)
    pltpu.sync_copy(idx_hbm.at[wid], idx_v)
    unpack(0, src_a, dst_a)
    unpack(1, src_b, dst_b)
    pltpu.make_async_copy(zero_hbm, acc_sh.at[pl.ds(r0, ROWS_PT)], sem_ga).wait()
    plsc.subcore_barrier()
    pltpu.async_copy(h_hbm.at[src_a], rows_a, sem_ga)
    pltpu.async_copy(h_hbm.at[src_b], rows_b, sem_gb)

    def body(j, carry):
        i0 = 2 * j
        pltpu.make_async_copy(h_hbm.at[src_a], rows_a, sem_ga).wait()
        pltpu.sync_copy(rows_a, acc_sh.at[dst_a], add=True)
        unpack(i0 + 2, src_a, dst_a)
        pltpu.async_copy(h_hbm.at[src_a], rows_a, sem_ga)

        pltpu.make_async_copy(h_hbm.at[src_b], rows_b, sem_gb).wait()
        pltpu.sync_copy(rows_b, acc_sh.at[dst_b], add=True)
        unpack(i0 + 3, src_b, dst_b)
        pltpu.async_copy(h_hbm.at[src_b], rows_b, sem_gb)
        return carry

    lax.fori_loop(0, NCHUNK // 2 - 1, body, 0)
    pltpu.make_async_copy(h_hbm.at[src_a], rows_a, sem_ga).wait()
    pltpu.sync_copy(rows_a, acc_sh.at[dst_a], add=True)
    pltpu.make_async_copy(h_hbm.at[src_b], rows_b, sem_gb).wait()
    pltpu.sync_copy(rows_b, acc_sh.at[dst_b], add=True)

    plsc.subcore_barrier()
    pltpu.sync_copy(acc_sh.at[pl.ds(r0, ROWS_PT)],
                    out_hbm.at[c, pl.ds(r0, ROWS_PT)])


def _res_body(h_ref, Wr_ref, br_ref, out_ref):
    r = jnp.dot(h_ref[...], Wr_ref[...], preferred_element_type=jnp.float32)
    out_ref[...] = jnp.maximum(r + br_ref[...], 0.0)


_res = pl.pallas_call(
    _res_body,
    out_shape=jax.ShapeDtypeStruct((N, H), jnp.float32),
)


def _layer_body(aggp_ref, res_ref, W_ref, b_ref, g_ref, be_ref, out_ref):
    agg = aggp_ref[0, :N, :] + aggp_ref[1, :N, :]
    o = jnp.dot(agg, W_ref[...], preferred_element_type=jnp.float32)
    o = jnp.maximum(o + b_ref[...], 0.0)
    o = o + res_ref[...]
    mu = jnp.mean(o, axis=0, keepdims=True)
    var = jnp.mean((o - mu) ** 2, axis=0, keepdims=True)
    out_ref[...] = g_ref[...] * (o - mu) / jnp.sqrt(var + 1e-5) + be_ref[...]


_layer = pl.pallas_call(
    _layer_body,
    out_shape=jax.ShapeDtypeStruct((N, H), jnp.float32),
)


def _layer2_readout_body(aggp_ref, res_ref, W_ref, b_ref,
                         g_ref, be_ref, watt_ref, batt_ref, out_ref):
    agg = aggp_ref[0, :N, :] + aggp_ref[1, :N, :]
    o = jnp.dot(agg, W_ref[...], preferred_element_type=jnp.float32)
    o = jnp.maximum(o + b_ref[...], 0.0)
    o = o + res_ref[...]
    mu = jnp.mean(o, axis=0, keepdims=True)
    var = jnp.mean((o - mu) ** 2, axis=0, keepdims=True)
    h2 = g_ref[...] * (o - mu) / jnp.sqrt(var + 1e-5) + be_ref[...]
    s = jnp.sum(h2 * watt_ref[...], axis=1, keepdims=True) + batt_ref[...]
    w = jax.nn.sigmoid(s)
    out_ref[:, :H] = jnp.sum(w * h2, axis=0, keepdims=True)
    out_ref[:, H:] = jnp.max(h2, axis=0, keepdims=True)


_layer2_readout = pl.pallas_call(
    _layer2_readout_body,
    out_shape=jax.ShapeDtypeStruct((1, 2 * H), jnp.float32),
)


def kernel(x, edge_index, W1, b1, Wr1, br1, g1, be1,
           W2, b2, Wr2, br2, g2, be2, w_att, b_att):
    npad_e = EPAD - E
    pad_idx = (jnp.arange(npad_e, dtype=jnp.int32) * 37) % N | (
        (jnp.arange(npad_e, dtype=jnp.int32) % (NPAD - N) + N) << 16)
    idx = jnp.concatenate(
        [edge_index[0] | (edge_index[1] << 16), pad_idx]
    ).reshape(NW, NCHUNK, CH)
    zeros = jnp.zeros((CH, H), jnp.float32)

    aggp1 = _segsum(x, idx, zeros)
    res1 = _res(x, Wr1, br1.reshape(1, H))
    h1 = _layer(aggp1, res1, W1, b1.reshape(1, H),
                g1.reshape(1, H), be1.reshape(1, H))
    aggp2 = _segsum(h1, idx, zeros)
    res2 = _res(h1, Wr2, br2.reshape(1, H))
    out = _layer2_readout(aggp2, res2, W2, b2.reshape(1, H),
                          g2.reshape(1, H),
                          be2.reshape(1, H), w_att.reshape(1, H),
                          b_att.reshape(1, 1))
    return out


# speedup vs baseline: 1.0445x; 1.0445x over previous
import functools

import jax
import jax.numpy as jnp
from jax import lax
from jax.experimental import pallas as pl
from jax.experimental.pallas import tpu as pltpu
from jax.experimental.pallas import tpu_sc as plsc

N = 10000
E = 320000
H = 128

NC = 2
NS = 16
NW = NC * NS
CH = 128
NCHUNK = 80
EPW = NCHUNK * CH
EPAD = NW * EPW
NPAD = 10240
ROWS_PT = NPAD // NS

_sc_mesh = plsc.VectorSubcoreMesh(core_axis_name="c", subcore_axis_name="s")


@functools.partial(
    pl.kernel,
    out_type=jax.ShapeDtypeStruct((NC, NPAD, H), jnp.float32),
    mesh=_sc_mesh,
    scratch_types=[
        pltpu.VMEM((NCHUNK, CH), jnp.int32),
        pltpu.VMEM((CH,), jnp.int32),
        pltpu.VMEM((CH,), jnp.int32),
        pltpu.VMEM((CH,), jnp.int32),
        pltpu.VMEM((CH,), jnp.int32),
        pltpu.VMEM((CH, H), jnp.float32),
        pltpu.VMEM((CH, H), jnp.float32),
        pltpu.VMEM_SHARED((NPAD, H), jnp.float32),
        pltpu.SemaphoreType.DMA,
        pltpu.SemaphoreType.DMA,
    ],
)
def _segsum(h_hbm, idx_hbm, zero_hbm, out_hbm,
            idx_v, src_a, dst_a, src_b, dst_b, rows_a, rows_b, acc_sh,
            sem_ga, sem_gb):
    c = lax.axis_index("c")
    s = lax.axis_index("s")
    wid = c * NS + s

    def unpack(i, src_ref, dst_ref):
        for k in range(CH // 16):
            v = idx_v[i, pl.ds(k * 16, 16)]
            src_ref[pl.ds(k * 16, 16)] = v & 0xFFFF
            dst_ref[pl.ds(k * 16, 16)] = lax.shift_right_logical(v, 16)

    r0 = s * ROWS_PT
    zdesc = pltpu.async_copy(zero_hbm.at[pl.ds(r0, ROWS_PT)],
                             acc_sh.at[pl.ds(r0, ROWS_PT)], sem_ga)
    pltpu.sync_copy(idx_hbm.at[wid], idx_v)
    unpack(0, src_a, dst_a)
    unpack(1, src_b, dst_b)
    zdesc.wait()
    plsc.subcore_barrier()
    pltpu.async_copy(h_hbm.at[src_a], rows_a, sem_ga)
    pltpu.async_copy(h_hbm.at[src_b], rows_b, sem_gb)

    def body(j, carry):
        i0 = 2 * j
        pltpu.make_async_copy(h_hbm.at[src_a], rows_a, sem_ga).wait()
        pltpu.sync_copy(rows_a, acc_sh.at[dst_a], add=True)
        unpack(i0 + 2, src_a, dst_a)
        pltpu.async_copy(h_hbm.at[src_a], rows_a, sem_ga)

        pltpu.make_async_copy(h_hbm.at[src_b], rows_b, sem_gb).wait()
        pltpu.sync_copy(rows_b, acc_sh.at[dst_b], add=True)
        unpack(i0 + 3, src_b, dst_b)
        pltpu.async_copy(h_hbm.at[src_b], rows_b, sem_gb)
        return carry

    lax.fori_loop(0, NCHUNK // 2 - 1, body, 0)
    pltpu.make_async_copy(h_hbm.at[src_a], rows_a, sem_ga).wait()
    pltpu.sync_copy(rows_a, acc_sh.at[dst_a], add=True)
    pltpu.make_async_copy(h_hbm.at[src_b], rows_b, sem_gb).wait()
    pltpu.sync_copy(rows_b, acc_sh.at[dst_b], add=True)

    plsc.subcore_barrier()
    pltpu.sync_copy(acc_sh.at[pl.ds(r0, ROWS_PT)],
                    out_hbm.at[c, pl.ds(r0, ROWS_PT)])


def _res_body(h_ref, Wr_ref, br_ref, out_ref):
    r = jnp.dot(h_ref[...], Wr_ref[...], preferred_element_type=jnp.float32)
    out_ref[...] = jnp.maximum(r + br_ref[...], 0.0)


_res = pl.pallas_call(
    _res_body,
    out_shape=jax.ShapeDtypeStruct((N, H), jnp.float32),
)


def _layer_body(aggp_ref, res_ref, W_ref, b_ref, g_ref, be_ref, out_ref):
    agg = aggp_ref[0, :N, :] + aggp_ref[1, :N, :]
    o = jnp.dot(agg, W_ref[...], preferred_element_type=jnp.float32)
    o = jnp.maximum(o + b_ref[...], 0.0)
    o = o + res_ref[...]
    mu = jnp.mean(o, axis=0, keepdims=True)
    var = jnp.mean((o - mu) ** 2, axis=0, keepdims=True)
    out_ref[...] = g_ref[...] * (o - mu) / jnp.sqrt(var + 1e-5) + be_ref[...]


_layer = pl.pallas_call(
    _layer_body,
    out_shape=jax.ShapeDtypeStruct((N, H), jnp.float32),
)


def _layer2_readout_body(aggp_ref, res_ref, W_ref, b_ref,
                         g_ref, be_ref, watt_ref, batt_ref, out_ref):
    agg = aggp_ref[0, :N, :] + aggp_ref[1, :N, :]
    o = jnp.dot(agg, W_ref[...], preferred_element_type=jnp.float32)
    o = jnp.maximum(o + b_ref[...], 0.0)
    o = o + res_ref[...]
    mu = jnp.mean(o, axis=0, keepdims=True)
    var = jnp.mean((o - mu) ** 2, axis=0, keepdims=True)
    h2 = g_ref[...] * (o - mu) / jnp.sqrt(var + 1e-5) + be_ref[...]
    s = jnp.sum(h2 * watt_ref[...], axis=1, keepdims=True) + batt_ref[...]
    w = jax.nn.sigmoid(s)
    out_ref[:, :H] = jnp.sum(w * h2, axis=0, keepdims=True)
    out_ref[:, H:] = jnp.max(h2, axis=0, keepdims=True)


_layer2_readout = pl.pallas_call(
    _layer2_readout_body,
    out_shape=jax.ShapeDtypeStruct((1, 2 * H), jnp.float32),
)


def kernel(x, edge_index, W1, b1, Wr1, br1, g1, be1,
           W2, b2, Wr2, br2, g2, be2, w_att, b_att):
    npad_e = EPAD - E
    ar = jnp.arange(npad_e, dtype=jnp.int32)
    pad_idx = ((ar * 37) % N | ((ar % (NPAD - N) + N) << 16)).reshape(-1, CH)
    ei = edge_index.reshape(2, E // CH, CH)
    idx = jnp.concatenate(
        [ei[0] | (ei[1] << 16), pad_idx], axis=0).reshape(NW, NCHUNK, CH)
    zeros = jnp.zeros((NPAD, H), jnp.float32)

    aggp1 = _segsum(x, idx, zeros)
    res1 = _res(x, Wr1, br1.reshape(1, H))
    h1 = _layer(aggp1, res1, W1, b1.reshape(1, H),
                g1.reshape(1, H), be1.reshape(1, H))
    aggp2 = _segsum(h1, idx, zeros)
    res2 = _res(h1, Wr2, br2.reshape(1, H))
    out = _layer2_readout(aggp2, res2, W2, b2.reshape(1, H),
                          g2.reshape(1, H),
                          be2.reshape(1, H), w_att.reshape(1, H),
                          b_att.reshape(1, 1))
    return out


# speedup vs baseline: 1.0452x; 1.0007x over previous
import functools

import jax
import jax.numpy as jnp
from jax import lax
from jax.experimental import pallas as pl
from jax.experimental.pallas import tpu as pltpu
from jax.experimental.pallas import tpu_sc as plsc

N = 10000
E = 320000
H = 128

NC = 2
NS = 16
NW = NC * NS
CH = 128
NCHUNK = 80
EPW = NCHUNK * CH
EPAD = NW * EPW
NPAD = 10240
ROWS_PT = NPAD // NS

_sc_mesh = plsc.VectorSubcoreMesh(core_axis_name="c", subcore_axis_name="s")


@functools.partial(
    pl.kernel,
    out_type=jax.ShapeDtypeStruct((NC, NPAD, H), jnp.float32),
    mesh=_sc_mesh,
    scratch_types=[
        pltpu.VMEM((NCHUNK // 8, 8, CH), jnp.int32),
        pltpu.VMEM((CH,), jnp.int32),
        pltpu.VMEM((CH,), jnp.int32),
        pltpu.VMEM((CH,), jnp.int32),
        pltpu.VMEM((CH,), jnp.int32),
        pltpu.VMEM((CH, H), jnp.float32),
        pltpu.VMEM((CH, H), jnp.float32),
        pltpu.VMEM_SHARED((NPAD, H), jnp.float32),
        pltpu.SemaphoreType.DMA,
        pltpu.SemaphoreType.DMA,
    ],
)
def _segsum(h_hbm, idx_hbm, zero_hbm, out_hbm,
            idx_v, src_a, dst_a, src_b, dst_b, rows_a, rows_b, acc_sh,
            sem_ga, sem_gb):
    c = lax.axis_index("c")
    s = lax.axis_index("s")
    wid = c * NS + s

    def unpack(i, src_ref, dst_ref):
        for k in range(CH // 16):
            v = idx_v[i // 8, i % 8, pl.ds(k * 16, 16)]
            src_ref[pl.ds(k * 16, 16)] = v & 0xFFFF
            dst_ref[pl.ds(k * 16, 16)] = lax.shift_right_logical(v, 16)

    r0 = s * ROWS_PT
    zdesc = pltpu.async_copy(zero_hbm.at[pl.ds(r0, ROWS_PT)],
                             acc_sh.at[pl.ds(r0, ROWS_PT)], sem_ga)
    pltpu.sync_copy(idx_hbm.at[:, wid], idx_v)
    unpack(0, src_a, dst_a)
    unpack(1, src_b, dst_b)
    zdesc.wait()
    plsc.subcore_barrier()
    pltpu.async_copy(h_hbm.at[src_a], rows_a, sem_ga)
    pltpu.async_copy(h_hbm.at[src_b], rows_b, sem_gb)

    def body(j, carry):
        i0 = 2 * j
        pltpu.make_async_copy(h_hbm.at[src_a], rows_a, sem_ga).wait()
        pltpu.sync_copy(rows_a, acc_sh.at[dst_a], add=True)
        unpack(i0 + 2, src_a, dst_a)
        pltpu.async_copy(h_hbm.at[src_a], rows_a, sem_ga)

        pltpu.make_async_copy(h_hbm.at[src_b], rows_b, sem_gb).wait()
        pltpu.sync_copy(rows_b, acc_sh.at[dst_b], add=True)
        unpack(i0 + 3, src_b, dst_b)
        pltpu.async_copy(h_hbm.at[src_b], rows_b, sem_gb)
        return carry

    lax.fori_loop(0, NCHUNK // 2 - 1, body, 0)
    pltpu.make_async_copy(h_hbm.at[src_a], rows_a, sem_ga).wait()
    pltpu.sync_copy(rows_a, acc_sh.at[dst_a], add=True)
    pltpu.make_async_copy(h_hbm.at[src_b], rows_b, sem_gb).wait()
    pltpu.sync_copy(rows_b, acc_sh.at[dst_b], add=True)

    plsc.subcore_barrier()
    pltpu.sync_copy(acc_sh.at[pl.ds(r0, ROWS_PT)],
                    out_hbm.at[c, pl.ds(r0, ROWS_PT)])


def _res_body(h_ref, Wr_ref, br_ref, out_ref):
    r = jnp.dot(h_ref[...], Wr_ref[...], preferred_element_type=jnp.float32)
    out_ref[...] = jnp.maximum(r + br_ref[...], 0.0)


_res = pl.pallas_call(
    _res_body,
    out_shape=jax.ShapeDtypeStruct((N, H), jnp.float32),
)


def _layer_body(aggp_ref, res_ref, W_ref, b_ref, g_ref, be_ref, out_ref):
    agg = aggp_ref[0, :N, :] + aggp_ref[1, :N, :]
    o = jnp.dot(agg, W_ref[...], preferred_element_type=jnp.float32)
    o = jnp.maximum(o + b_ref[...], 0.0)
    o = o + res_ref[...]
    mu = jnp.mean(o, axis=0, keepdims=True)
    var = jnp.mean((o - mu) ** 2, axis=0, keepdims=True)
    out_ref[...] = g_ref[...] * (o - mu) / jnp.sqrt(var + 1e-5) + be_ref[...]


_layer = pl.pallas_call(
    _layer_body,
    out_shape=jax.ShapeDtypeStruct((N, H), jnp.float32),
)


def _layer2_readout_body(aggp_ref, res_ref, W_ref, b_ref,
                         g_ref, be_ref, watt_ref, batt_ref, out_ref):
    agg = aggp_ref[0, :N, :] + aggp_ref[1, :N, :]
    o = jnp.dot(agg, W_ref[...], preferred_element_type=jnp.float32)
    o = jnp.maximum(o + b_ref[...], 0.0)
    o = o + res_ref[...]
    mu = jnp.mean(o, axis=0, keepdims=True)
    var = jnp.mean((o - mu) ** 2, axis=0, keepdims=True)
    h2 = g_ref[...] * (o - mu) / jnp.sqrt(var + 1e-5) + be_ref[...]
    s = jnp.sum(h2 * watt_ref[...], axis=1, keepdims=True) + batt_ref[...]
    w = jax.nn.sigmoid(s)
    out_ref[:, :H] = jnp.sum(w * h2, axis=0, keepdims=True)
    out_ref[:, H:] = jnp.max(h2, axis=0, keepdims=True)


_layer2_readout = pl.pallas_call(
    _layer2_readout_body,
    out_shape=jax.ShapeDtypeStruct((1, 2 * H), jnp.float32),
)


def kernel(x, edge_index, W1, b1, Wr1, br1, g1, be1,
           W2, b2, Wr2, br2, g2, be2, w_att, b_att):
    npad_e = EPAD - E
    ar = jnp.arange(npad_e, dtype=jnp.int32)
    pad_idx = ((ar * 37) % N | ((ar % (NPAD - N) + N) << 16)).reshape(-1, CH)
    ei = edge_index.reshape(2, E // CH, CH)
    idx = jnp.concatenate(
        [ei[0] | (ei[1] << 16), pad_idx], axis=0
    ).reshape(NCHUNK // 8, NW, 8, CH)
    zeros = jnp.zeros((NPAD, H), jnp.float32)

    aggp1 = _segsum(x, idx, zeros)
    res1 = _res(x, Wr1, br1.reshape(1, H))
    h1 = _layer(aggp1, res1, W1, b1.reshape(1, H),
                g1.reshape(1, H), be1.reshape(1, H))
    aggp2 = _segsum(h1, idx, zeros)
    res2 = _res(h1, Wr2, br2.reshape(1, H))
    out = _layer2_readout(aggp2, res2, W2, b2.reshape(1, H),
                          g2.reshape(1, H),
                          be2.reshape(1, H), w_att.reshape(1, H),
                          b_att.reshape(1, 1))
    return out
